# baseline (speedup 1.0000x reference)
import functools

import jax
import jax.numpy as jnp
from jax.experimental import pallas as pl
from jax.experimental.pallas import tpu as pltpu

_NUM_EMBED = 1024
_DIM = 64
_COMMIT = 0.25
_INTERPRET = False


def _vq_body(z_ref, w_ref, qst_ref, idx_ref, loss_ref, *, tile):
    z = z_ref[...]
    w = w_ref[...]
    zsq = jnp.sum(z * z, axis=1, keepdims=True)
    wsq = jnp.sum(w * w, axis=1)
    mm = jax.lax.dot_general(z, w, (((1,), (1,)), ((), ())),
                             preferred_element_type=jnp.float32)
    d = zsq + wsq[None, :] - 2.0 * mm
    dmin = jnp.min(d, axis=1, keepdims=True)
    col = jax.lax.broadcasted_iota(jnp.int32, d.shape, 1)
    idx = jnp.min(jnp.where(d == dmin, col, _NUM_EMBED), axis=1)
    onehot = (col == idx[:, None]).astype(jnp.float32)
    q = jax.lax.dot_general(onehot, w, (((1,), (0,)), ((), ())),
                            preferred_element_type=jnp.float32)
    diff = q - z
    qst_ref[...] = z + diff
    idx_ref[0, 0, :] = idx
    part = (1.0 + _COMMIT) * jnp.sum(diff * diff)

    @pl.when(pl.program_id(0) == 0)
    def _init():
        loss_ref[0, 0] = 0.0

    loss_ref[0, 0] += part


@functools.partial(jax.jit, static_argnames=("tile",))
def _vq_forward(flat, weight, tile):
    n = flat.shape[0]
    grid = n // tile
    qst, idx3, loss = pl.pallas_call(
        functools.partial(_vq_body, tile=tile),
        grid=(grid,),
        in_specs=[
            pl.BlockSpec((tile, _DIM), lambda i: (i, 0)),
            pl.BlockSpec((_NUM_EMBED, _DIM), lambda i: (0, 0)),
        ],
        out_specs=[
            pl.BlockSpec((tile, _DIM), lambda i: (i, 0)),
            pl.BlockSpec((1, 1, tile), lambda i: (i, 0, 0)),
            pl.BlockSpec(memory_space=pltpu.SMEM, block_shape=(1, 1),
                         index_map=lambda i: (0, 0)),
        ],
        out_shape=[
            jax.ShapeDtypeStruct((n, _DIM), jnp.float32),
            jax.ShapeDtypeStruct((grid, 1, tile), jnp.int32),
            jax.ShapeDtypeStruct((1, 1), jnp.float32),
        ],
        interpret=_INTERPRET,
    )(flat, weight)
    return qst, idx3.reshape(n), loss[0, 0] / (n * _DIM)


def kernel(inputs, embedding_weight):
    b, s, dim = inputs.shape
    flat = inputs.reshape(-1, dim)
    qst, idx, vq_loss = _vq_forward(flat, embedding_weight, tile=1152)
    return qst.reshape(inputs.shape), vq_loss, idx.reshape(b, s)
